# Initial kernel scaffold; baseline (speedup 1.0000x reference)
import jax
import jax.numpy as jnp
from jax.experimental import pallas as pl


def kernel(stage_indices, codebooks):
    raise NotImplementedError("write your pallas kernel here")


# speedup vs baseline: 7.4020x; 7.4020x over previous
import functools

import jax
import jax.numpy as jnp
from jax import lax
from jax.experimental import pallas as pl
from jax.experimental.pallas import tpu as pltpu
from jax.experimental.pallas import tpu_sc as plsc

NC = 2
NS = 16
NW = NC * NS
TC_CHUNK = 64


def _sc_decode(widx, cb_flat, S, D, P):
    NCH = P // TC_CHUNK
    mesh = plsc.VectorSubcoreMesh(core_axis_name="c", subcore_axis_name="s")

    @functools.partial(
        pl.kernel,
        out_type=jax.ShapeDtypeStruct((NW * P, D), jnp.float32),
        mesh=mesh,
        scratch_types=[
            pltpu.VMEM((NCH * S, TC_CHUNK), jnp.int32),
            pltpu.VMEM((TC_CHUNK, D), jnp.float32),
            pltpu.VMEM((TC_CHUNK, D), jnp.float32),
            pltpu.SemaphoreType.DMA,
        ],
    )
    def sc_decode(idx_hbm, cb_hbm, out_hbm, idx_v, acc_v, row_v, gsem):
        w = lax.axis_index("s") * NC + lax.axis_index("c")
        pltpu.sync_copy(idx_hbm.at[w], idx_v)

        def chunk_body(c, carry):
            base = c * S
            pltpu.async_copy(cb_hbm.at[idx_v.at[base]], acc_v, gsem).wait()
            for s in range(1, S):
                pltpu.async_copy(
                    cb_hbm.at[idx_v.at[base + s]], row_v, gsem
                ).wait()

                def add_body(t, carry2):
                    for dj in range(D // 16):
                        sl = pl.ds(dj * 16, 16)
                        plsc.addupdate(acc_v.at[t, sl], row_v[t, sl])
                    return carry2

                lax.fori_loop(0, TC_CHUNK, add_body, 0)
            pltpu.sync_copy(
                acc_v, out_hbm.at[pl.ds(w * P + c * TC_CHUNK, TC_CHUNK)]
            )
            return carry

        lax.fori_loop(0, NCH, chunk_body, 0)

    return sc_decode(widx, cb_flat)


def _tc_transpose(tmp, B, T, D):
    TT = 256

    def body(x_ref, o_ref):
        o_ref[0] = jnp.swapaxes(x_ref[0], 0, 1)

    return pl.pallas_call(
        body,
        grid=(B, T // TT),
        in_specs=[pl.BlockSpec((1, TT, D), lambda b, t: (b, t, 0))],
        out_specs=pl.BlockSpec((1, D, TT), lambda b, t: (b, 0, t)),
        out_shape=jax.ShapeDtypeStruct((B, D, T), jnp.float32),
    )(tmp)


def kernel(stage_indices, codebooks):
    S, K, D = codebooks.shape
    _, B, T = stage_indices.shape
    P = B * T // NW
    NCH = P // TC_CHUNK

    cb_flat = codebooks.reshape(S * K, D)
    idx = stage_indices.astype(jnp.int32) + (
        jnp.arange(S, dtype=jnp.int32) * K
    )[:, None, None]
    widx = (
        idx.transpose(1, 2, 0)
        .reshape(NW, NCH, TC_CHUNK, S)
        .transpose(0, 1, 3, 2)
        .reshape(NW, NCH * S, TC_CHUNK)
    )

    tmp = _sc_decode(widx, cb_flat, S, D, P)
    return _tc_transpose(tmp.reshape(B, T, D), B, T, D)


# speedup vs baseline: 18.3529x; 2.4795x over previous
import functools

import jax
import jax.numpy as jnp
from jax import lax
from jax.experimental import pallas as pl
from jax.experimental.pallas import tpu as pltpu
from jax.experimental.pallas import tpu_sc as plsc

NC = 2
NS = 16
NW = NC * NS
TCH = 16


def _sc_decode(widx, cb_flat, S, D, P):
    NCH = P // TCH
    mesh = plsc.VectorSubcoreMesh(core_axis_name="c", subcore_axis_name="s")

    @functools.partial(
        pl.kernel,
        out_type=jax.ShapeDtypeStruct((NW * P, D), jnp.float32),
        mesh=mesh,
        scratch_types=[
            pltpu.VMEM((NCH * S, TCH), jnp.int32),
            pltpu.VMEM((2, S * TCH, D), jnp.float32),
            pltpu.SemaphoreType.DMA,
            pltpu.SemaphoreType.DMA,
        ],
    )
    def sc_decode(idx_hbm, cb_hbm, out_hbm, idx_v, sbuf, sem0, sem1):
        sems = (sem0, sem1)
        w = lax.axis_index("s") * NC + lax.axis_index("c")
        pltpu.sync_copy(idx_hbm.at[w], idx_v)

        def fire(c, par):
            for s in range(S):
                pltpu.async_copy(
                    cb_hbm.at[idx_v.at[c * S + s]],
                    sbuf.at[par, pl.ds(s * TCH, TCH)],
                    sems[par],
                )

        def drain(par):
            pltpu.make_async_copy(
                cb_hbm.at[pl.ds(0, S * TCH)], sbuf.at[par], sems[par]
            ).wait()

        fire(0, 0)

        def outer(cc, carry):
            for par in range(2):
                c = cc * 2 + par
                cn = jnp.minimum(c + 1, NCH - 1)
                fire(cn, 1 - par)
                drain(par)

                @plsc.parallel_loop(0, TCH, unroll=2)
                def t_body(t):
                    for dj in range(D // 16):
                        sl = pl.ds(dj * 16, 16)
                        v01 = sbuf[par, t, sl] + sbuf[par, TCH + t, sl]
                        v23 = (
                            sbuf[par, 2 * TCH + t, sl]
                            + sbuf[par, 3 * TCH + t, sl]
                        )
                        v45 = (
                            sbuf[par, 4 * TCH + t, sl]
                            + sbuf[par, 5 * TCH + t, sl]
                        )
                        v67 = (
                            sbuf[par, 6 * TCH + t, sl]
                            + sbuf[par, 7 * TCH + t, sl]
                        )
                        sbuf[par, t, sl] = (v01 + v23) + (v45 + v67)

                pltpu.sync_copy(
                    sbuf.at[par, pl.ds(0, TCH)],
                    out_hbm.at[pl.ds(w * P + c * TCH, TCH)],
                )
            return carry

        lax.fori_loop(0, NCH // 2, outer, 0)
        drain(0)

    return sc_decode(widx, cb_flat)


def _tc_transpose(tmp, B, T, D):
    TT = 256

    def body(x_ref, o_ref):
        o_ref[0] = jnp.swapaxes(x_ref[0], 0, 1)

    return pl.pallas_call(
        body,
        grid=(B, T // TT),
        in_specs=[pl.BlockSpec((1, TT, D), lambda b, t: (b, t, 0))],
        out_specs=pl.BlockSpec((1, D, TT), lambda b, t: (b, 0, t)),
        out_shape=jax.ShapeDtypeStruct((B, D, T), jnp.float32),
    )(tmp)


def kernel(stage_indices, codebooks):
    S, K, D = codebooks.shape
    _, B, T = stage_indices.shape
    P = B * T // NW
    NCH = P // TCH

    cb_flat = codebooks.reshape(S * K, D)
    idx = stage_indices.astype(jnp.int32) + (
        jnp.arange(S, dtype=jnp.int32) * K
    )[:, None, None]
    widx = (
        idx.transpose(1, 2, 0)
        .reshape(NW, NCH, TCH, S)
        .transpose(0, 1, 3, 2)
        .reshape(NW, NCH * S, TCH)
    )

    tmp = _sc_decode(widx, cb_flat, S, D, P)
    return _tc_transpose(tmp.reshape(B, T, D), B, T, D)


# speedup vs baseline: 22.9545x; 1.2507x over previous
import functools

import jax
import jax.numpy as jnp
from jax import lax
from jax.experimental import pallas as pl
from jax.experimental.pallas import tpu as pltpu
from jax.experimental.pallas import tpu_sc as plsc

NC = 2
NS = 16
NW = NC * NS
TCH = 16


def _sc_decode(widx, cb_flat, S, D, P):
    NCH = P // TCH
    mesh = plsc.VectorSubcoreMesh(core_axis_name="c", subcore_axis_name="s")

    @functools.partial(
        pl.kernel,
        out_type=jax.ShapeDtypeStruct((NW * P, D), jnp.float32),
        mesh=mesh,
        scratch_types=[
            pltpu.VMEM((NCH * S, TCH), jnp.int32),
            pltpu.VMEM((2, S * TCH, D), jnp.float32),
            pltpu.SemaphoreType.DMA,
            pltpu.SemaphoreType.DMA,
        ],
    )
    def sc_decode(idx_hbm, cb_hbm, out_hbm, idx_v, sbuf, sem0, sem1):
        sems = (sem0, sem1)
        w = lax.axis_index("s") * NC + lax.axis_index("c")
        pltpu.sync_copy(idx_hbm.at[w], idx_v)

        def fire(c, par):
            for s in range(S):
                pltpu.async_copy(
                    cb_hbm.at[idx_v.at[c * S + s]],
                    sbuf.at[par, pl.ds(s * TCH, TCH)],
                    sems[par],
                )

        def drain(par):
            pltpu.make_async_copy(
                cb_hbm.at[pl.ds(0, S * TCH)], sbuf.at[par], sems[par]
            ).wait()

        fire(0, 0)

        def outer(cc, carry):
            for par in range(2):
                c = cc * 2 + par
                cn = jnp.minimum(c + 1, NCH - 1)
                fire(cn, 1 - par)
                drain(par)

                @plsc.parallel_loop(0, TCH, unroll=2)
                def t_body(t):
                    for dj in range(D // 16):
                        sl = pl.ds(dj * 16, 16)
                        v01 = sbuf[par, t, sl] + sbuf[par, TCH + t, sl]
                        v23 = (
                            sbuf[par, 2 * TCH + t, sl]
                            + sbuf[par, 3 * TCH + t, sl]
                        )
                        v45 = (
                            sbuf[par, 4 * TCH + t, sl]
                            + sbuf[par, 5 * TCH + t, sl]
                        )
                        v67 = (
                            sbuf[par, 6 * TCH + t, sl]
                            + sbuf[par, 7 * TCH + t, sl]
                        )
                        sbuf[par, t, sl] = (v01 + v23) + (v45 + v67)

                pltpu.sync_copy(
                    sbuf.at[par, pl.ds(0, TCH)],
                    out_hbm.at[pl.ds(w * P + c * TCH, TCH)],
                )
            return carry

        lax.fori_loop(0, NCH // 2, outer, 0)
        drain(0)

    return sc_decode(widx, cb_flat)


def _tc_transpose(tmp, B, T, D):
    def body(x_ref, o_ref):
        o_ref[0] = jnp.swapaxes(x_ref[0], 0, 1)

    return pl.pallas_call(
        body,
        grid=(B,),
        in_specs=[pl.BlockSpec((1, T, D), lambda b: (b, 0, 0))],
        out_specs=pl.BlockSpec((1, D, T), lambda b: (b, 0, 0)),
        out_shape=jax.ShapeDtypeStruct((B, D, T), jnp.float32),
    )(tmp)


def kernel(stage_indices, codebooks):
    S, K, D = codebooks.shape
    _, B, T = stage_indices.shape
    P = B * T // NW
    NCH = P // TCH

    cb_flat = codebooks.reshape(S * K, D)
    idx = stage_indices.astype(jnp.int32) + (
        jnp.arange(S, dtype=jnp.int32) * K
    )[:, None, None]
    widx = (
        idx.transpose(1, 2, 0)
        .reshape(NW, NCH, TCH, S)
        .transpose(0, 1, 3, 2)
        .reshape(NW, NCH * S, TCH)
    )

    tmp = _sc_decode(widx, cb_flat, S, D, P)
    return _tc_transpose(tmp.reshape(B, T, D), B, T, D)


# speedup vs baseline: 23.3096x; 1.0155x over previous
import functools

import jax
import jax.numpy as jnp
from jax import lax
from jax.experimental import pallas as pl
from jax.experimental.pallas import tpu as pltpu
from jax.experimental.pallas import tpu_sc as plsc

NC = 2
NS = 16
NW = NC * NS
TCH = 16


def _sc_decode(widx, cb_flat, S, D, P):
    NCH = P // TCH
    mesh = plsc.VectorSubcoreMesh(core_axis_name="c", subcore_axis_name="s")

    @functools.partial(
        pl.kernel,
        out_type=jax.ShapeDtypeStruct((NW * P, D), jnp.float32),
        mesh=mesh,
        scratch_types=[
            pltpu.VMEM((NCH * S, TCH), jnp.int32),
            pltpu.VMEM((2, S * TCH, D), jnp.float32),
            pltpu.SemaphoreType.DMA,
            pltpu.SemaphoreType.DMA,
            pltpu.SemaphoreType.DMA,
            pltpu.SemaphoreType.DMA,
        ],
    )
    def sc_decode(
        idx_hbm, cb_hbm, out_hbm, idx_v, sbuf, sem0, sem1, osem0, osem1
    ):
        sems = (sem0, sem1)
        osems = (osem0, osem1)
        w = lax.axis_index("s") * NC + lax.axis_index("c")
        pltpu.sync_copy(idx_hbm.at[w], idx_v)

        def fire(c, par):
            for s in range(S):
                pltpu.async_copy(
                    cb_hbm.at[idx_v.at[c * S + s]],
                    sbuf.at[par, pl.ds(s * TCH, TCH)],
                    sems[par],
                )

        def drain(par):
            pltpu.make_async_copy(
                cb_hbm.at[pl.ds(0, S * TCH)], sbuf.at[par], sems[par]
            ).wait()

        def wait_out(par):
            pltpu.make_async_copy(
                sbuf.at[par, pl.ds(0, TCH)],
                out_hbm.at[pl.ds(0, TCH)],
                osems[par],
            ).wait()

        fire(0, 0)

        def outer(cc, carry):
            for par in range(2):
                c = cc * 2 + par
                cn = jnp.minimum(c + 1, NCH - 1)
                if par == 0:
                    @pl.when(cc > 0)
                    def _():
                        wait_out(1 - par)
                else:
                    wait_out(1 - par)
                fire(cn, 1 - par)
                drain(par)

                @plsc.parallel_loop(0, TCH, unroll=2)
                def t_body(t):
                    for dj in range(D // 16):
                        sl = pl.ds(dj * 16, 16)
                        v01 = sbuf[par, t, sl] + sbuf[par, TCH + t, sl]
                        v23 = (
                            sbuf[par, 2 * TCH + t, sl]
                            + sbuf[par, 3 * TCH + t, sl]
                        )
                        v45 = (
                            sbuf[par, 4 * TCH + t, sl]
                            + sbuf[par, 5 * TCH + t, sl]
                        )
                        v67 = (
                            sbuf[par, 6 * TCH + t, sl]
                            + sbuf[par, 7 * TCH + t, sl]
                        )
                        sbuf[par, t, sl] = (v01 + v23) + (v45 + v67)

                pltpu.async_copy(
                    sbuf.at[par, pl.ds(0, TCH)],
                    out_hbm.at[pl.ds(w * P + c * TCH, TCH)],
                    osems[par],
                )
            return carry

        lax.fori_loop(0, NCH // 2, outer, 0)
        drain(0)
        wait_out(1)

    return sc_decode(widx, cb_flat)


def _tc_transpose(tmp, B, T, D):
    def body(x_ref, o_ref):
        o_ref[0] = jnp.swapaxes(x_ref[0], 0, 1)

    return pl.pallas_call(
        body,
        grid=(B,),
        in_specs=[pl.BlockSpec((1, T, D), lambda b: (b, 0, 0))],
        out_specs=pl.BlockSpec((1, D, T), lambda b: (b, 0, 0)),
        out_shape=jax.ShapeDtypeStruct((B, D, T), jnp.float32),
    )(tmp)


def kernel(stage_indices, codebooks):
    S, K, D = codebooks.shape
    _, B, T = stage_indices.shape
    P = B * T // NW
    NCH = P // TCH

    cb_flat = codebooks.reshape(S * K, D)
    idx = stage_indices.astype(jnp.int32) + (
        jnp.arange(S, dtype=jnp.int32) * K
    )[:, None, None]
    widx = (
        idx.transpose(1, 2, 0)
        .reshape(NW, NCH, TCH, S)
        .transpose(0, 1, 3, 2)
        .reshape(NW, NCH * S, TCH)
    )

    tmp = _sc_decode(widx, cb_flat, S, D, P)
    return _tc_transpose(tmp.reshape(B, T, D), B, T, D)
